# Initial kernel scaffold; baseline (speedup 1.0000x reference)
import jax
import jax.numpy as jnp
from jax.experimental import pallas as pl


def kernel(tokens, word_embeddings):
    raise NotImplementedError("write your pallas kernel here")


# speedup vs baseline: 1.5063x; 1.5063x over previous
import functools

import jax
import jax.numpy as jnp
from jax import lax
from jax.experimental import pallas as pl
from jax.experimental.pallas import tpu as pltpu
from jax.experimental.pallas import tpu_sc as plsc

VOCAB = 30522
EMBED_DIM = 768
NUM_TOKENS = 4 * 8192

NUM_CORES = 2
NUM_SUBCORES = 16
NUM_WORKERS = NUM_CORES * NUM_SUBCORES
TOK_PER_W = NUM_TOKENS // NUM_WORKERS
CHUNK = 64
NCHUNK = TOK_PER_W // CHUNK


def _emb_body(tok_hbm, tab_hbm, out_hbm, idx_v, rows_v, sem):
    wid = lax.axis_index("s") * NUM_CORES + lax.axis_index("c")
    base = wid * TOK_PER_W
    pltpu.sync_copy(tok_hbm.at[pl.ds(base, TOK_PER_W)], idx_v)
    for c in range(NCHUNK):
        idx_slice = idx_v.at[pl.ds(c * CHUNK, CHUNK)]
        pltpu.async_copy(tab_hbm.at[idx_slice], rows_v, sem).wait()
        pltpu.sync_copy(rows_v, out_hbm.at[pl.ds(base + c * CHUNK, CHUNK)])


@jax.jit
def _emb(tokens_flat, word_embeddings):
    mesh = plsc.VectorSubcoreMesh(
        core_axis_name="c",
        subcore_axis_name="s",
        num_cores=NUM_CORES,
        num_subcores=NUM_SUBCORES,
    )
    return pl.kernel(
        _emb_body,
        out_type=jax.ShapeDtypeStruct((NUM_TOKENS, EMBED_DIM), jnp.float32),
        mesh=mesh,
        scratch_types=[
            pltpu.VMEM((TOK_PER_W,), jnp.int32),
            pltpu.VMEM((CHUNK, EMBED_DIM), jnp.float32),
            pltpu.SemaphoreType.DMA,
        ],
    )(tokens_flat, word_embeddings)


def kernel(tokens, word_embeddings):
    b, s = tokens.shape
    flat = tokens.reshape(b * s).astype(jnp.int32)
    out = _emb(flat, word_embeddings)
    return out.reshape(b, s, EMBED_DIM)


# speedup vs baseline: 1.6768x; 1.1133x over previous
import functools

import jax
import jax.numpy as jnp
from jax import lax
from jax.experimental import pallas as pl
from jax.experimental.pallas import tpu as pltpu
from jax.experimental.pallas import tpu_sc as plsc

VOCAB = 30522
EMBED_DIM = 768
NUM_TOKENS = 4 * 8192

NUM_CORES = 2
NUM_SUBCORES = 16
NUM_WORKERS = NUM_CORES * NUM_SUBCORES
TOK_PER_W = NUM_TOKENS // NUM_WORKERS
CHUNK = 64
NCHUNK = TOK_PER_W // CHUNK


def _emb_body(tok_hbm, tab_hbm, out_hbm, idx_v, rows0, rows1, g0, g1, o0, o1):
    wid = lax.axis_index("s") * NUM_CORES + lax.axis_index("c")
    base = wid * TOK_PER_W
    pltpu.sync_copy(tok_hbm.at[pl.ds(base, TOK_PER_W)], idx_v)
    rows = (rows0, rows1)
    gsem = (g0, g1)
    osem = (o0, o1)

    def gather(c):
        idx_slice = idx_v.at[pl.ds(c * CHUNK, CHUNK)]
        return pltpu.async_copy(tab_hbm.at[idx_slice], rows[c & 1], gsem[c & 1])

    def put(c):
        dst = out_hbm.at[pl.ds(base + c * CHUNK, CHUNK)]
        return pltpu.async_copy(rows[c & 1], dst, osem[c & 1])

    gathers = {0: gather(0)}
    puts = {}
    for c in range(NCHUNK):
        if c + 1 < NCHUNK:
            if c >= 1:
                puts[c - 1].wait()
            gathers[c + 1] = gather(c + 1)
        gathers[c].wait()
        puts[c] = put(c)
    puts[NCHUNK - 2].wait()
    puts[NCHUNK - 1].wait()


@jax.jit
def _emb(tokens_flat, word_embeddings):
    mesh = plsc.VectorSubcoreMesh(
        core_axis_name="c",
        subcore_axis_name="s",
        num_cores=NUM_CORES,
        num_subcores=NUM_SUBCORES,
    )
    return pl.kernel(
        _emb_body,
        out_type=jax.ShapeDtypeStruct((NUM_TOKENS, EMBED_DIM), jnp.float32),
        mesh=mesh,
        scratch_types=[
            pltpu.VMEM((TOK_PER_W,), jnp.int32),
            pltpu.VMEM((CHUNK, EMBED_DIM), jnp.float32),
            pltpu.VMEM((CHUNK, EMBED_DIM), jnp.float32),
            pltpu.SemaphoreType.DMA,
            pltpu.SemaphoreType.DMA,
            pltpu.SemaphoreType.DMA,
            pltpu.SemaphoreType.DMA,
        ],
    )(tokens_flat, word_embeddings)


def kernel(tokens, word_embeddings):
    b, s = tokens.shape
    flat = tokens.reshape(b * s).astype(jnp.int32)
    out = _emb(flat, word_embeddings)
    return out.reshape(b, s, EMBED_DIM)
